# baseline (speedup 1.0000x reference)
import functools
import jax
import jax.numpy as jnp
from jax import lax
from jax.experimental import pallas as pl
from jax.experimental.pallas import tpu as pltpu
from jax.experimental.pallas import tpu_sc as plsc

H = 128
LANES = 16
HV = H // LANES
NC = 2
NS = 16
NW = NC * NS
CH = 64
NBUF = 5
EPS = 1e-12


def _rsqrt(v):
    bits = lax.bitcast_convert_type(v, jnp.int32)
    y = lax.bitcast_convert_type(
        jnp.int32(0x5F3759DF) - lax.shift_right_logical(bits, 1), jnp.float32)
    half = 0.5 * v
    for _ in range(3):
        y = y * (1.5 - half * y * y)
    return y


_GATHER_DNUMS = lax.GatherDimensionNumbers(
    offset_dims=(), collapsed_slice_dims=(0,), start_index_map=(0,))


def _lane_shuffle(x, idx):
    return lax.gather(x, idx[:, None], _GATHER_DNUMS, (1,),
                      mode=lax.GatherScatterMode.PROMISE_IN_BOUNDS)


def _hsum(x):
    lanes = lax.iota(jnp.int32, LANES)
    for sh in (8, 4, 2, 1):
        x = x + _lane_shuffle(x, lanes ^ sh)
    return x


def _sc_fused(tgt_ids2d, msa_ids2d, table, pos_tab, gamma, beta, B, L, n_msa):
    total = B * L
    P = total // NW
    n_chunks = P // CH
    n_steps = n_chunks * n_msa
    inv_n = 1.0 / n_msa
    inv_h = 1.0 / H

    mesh = plsc.VectorSubcoreMesh(core_axis_name="c", subcore_axis_name="s")

    @functools.partial(
        pl.kernel,
        out_type=(
            jax.ShapeDtypeStruct((total, H), jnp.float32),
            jax.ShapeDtypeStruct((total, H), jnp.float32),
        ),
        mesh=mesh,
        scratch_types=[
            pltpu.VMEM((B, P), jnp.int32),
            pltpu.VMEM((P // 128, n_msa, 128), jnp.int32),
            pltpu.VMEM((P, H), jnp.float32),
            pltpu.VMEM((NBUF, CH, H), jnp.float32),
            pltpu.VMEM((P, H), jnp.float32),
            pltpu.VMEM((2, CH, H), jnp.float32),
            pltpu.VMEM((H,), jnp.float32),
            pltpu.VMEM((H,), jnp.float32),
            pltpu.SemaphoreType.DMA,
            pltpu.SemaphoreType.DMA,
            pltpu.SemaphoreType.DMA,
            pltpu.SemaphoreType.DMA,
            pltpu.SemaphoreType.DMA((NBUF,)),
            pltpu.SemaphoreType.DMA((2,)),
            pltpu.SemaphoreType.DMA,
        ],
    )
    def k(tgt_hbm, msa_hbm, table_hbm, pos_hbm, gamma_hbm, beta_hbm,
          emb_out, m_out,
          tidx, midx, trows, ring, acc, pbuf, gvec, bvec,
          sem_im, sem_i, sem_t, sem_a, rsems, psems, sem_o):
        wid = lax.axis_index("s") * NC + lax.axis_index("c")
        base = pl.multiple_of(wid * P, P)
        b = base // L
        l0 = pl.multiple_of(base - b * L, P)

        mcps = []
        for t in range(P // 128):
            mcps.append(pltpu.make_async_copy(
                msa_hbm.at[pl.ds(b * n_msa, n_msa), pl.ds(l0 + t * 128, 128)],
                midx.at[t], sem_im))
        for cp in mcps:
            cp.start()
        icps = [
            pltpu.make_async_copy(tgt_hbm.at[:, pl.ds(l0, P)], tidx, sem_i),
            pltpu.make_async_copy(gamma_hbm, gvec, sem_i),
            pltpu.make_async_copy(beta_hbm, bvec, sem_i),
        ]
        for cp in icps:
            cp.start()

        for ci in range(min(2, n_chunks)):
            pltpu.make_async_copy(
                pos_hbm.at[pl.ds(l0 + ci * CH, CH)], pbuf.at[ci % 2],
                psems.at[ci % 2]).start()

        def _mo(x, m):
            return x if isinstance(x, int) else pl.multiple_of(x, m)

        def gather_cp(step):
            ci = step // n_msa
            j = step - ci * n_msa
            t = ci // 2
            half = ci - t * 2
            isl = midx.at[t, j, pl.ds(_mo(half * CH, CH), CH)]
            a0 = _mo(ci * CH, CH)

            def to_acc():
                return pltpu.make_async_copy(
                    table_hbm.at[isl], acc.at[pl.ds(a0, CH)], sem_a)

            def to_ring():
                sl = step % NBUF
                return pltpu.make_async_copy(
                    table_hbm.at[isl], ring.at[sl], rsems.at[sl])

            return j, ci, to_acc, to_ring

        def fire(step):
            j, _, to_acc, to_ring = gather_cp(step)
            if isinstance(step, int):
                (to_acc() if j == 0 else to_ring()).start()
                return

            @pl.when(j == 0)
            def _():
                to_acc().start()

            @pl.when(j != 0)
            def _():
                to_ring().start()

        def step_body(step, _):
            j, ci, to_acc, to_ring = gather_cp(step)
            a0 = _mo(ci * CH, CH)

            @pl.when(j == 0)
            def _():
                to_acc().wait()

            @pl.when(j != 0)
            def _():
                to_ring().wait()
                sl = step % NBUF
                src = ring.at[sl]

                @plsc.parallel_loop(0, CH, unroll=2)
                def add_row(p):
                    for h in range(HV):
                        hs = pl.ds(h * LANES, LANES)
                        plsc.addupdate(acc.at[a0 + p, hs], src[p, hs])

            @pl.when(step + NBUF < n_steps)
            def _():
                fire(step + NBUF)

            @pl.when(j == n_msa - 1)
            def _():
                pltpu.make_async_copy(
                    table_hbm.at[tidx.at[b, pl.ds(a0, CH)]],
                    trows.at[pl.ds(a0, CH)], sem_t).wait()
                pb = ci % 2
                pltpu.make_async_copy(
                    pos_hbm.at[pl.ds(l0 + a0, CH)], pbuf.at[pb],
                    psems.at[pb]).wait()

                @plsc.parallel_loop(0, CH, unroll=2)
                def fin_row(p):
                    r = a0 + p
                    s16 = jnp.zeros((LANES,), jnp.float32)
                    q16 = jnp.zeros((LANES,), jnp.float32)
                    for h in range(HV):
                        hs = pl.ds(h * LANES, LANES)
                        m = acc[r, hs] * inv_n
                        acc[r, hs] = m
                        x = trows[r, hs] + m + pbuf[pb, p, hs]
                        trows[r, hs] = x
                        s16 = s16 + x
                        q16 = q16 + x * x
                    mean = _hsum(s16) * inv_h
                    var = _hsum(q16) * inv_h - mean * mean
                    inv = _rsqrt(var + EPS)
                    for h in range(HV):
                        hs = pl.ds(h * LANES, LANES)
                        trows[r, hs] = ((trows[r, hs] - mean) * inv
                                        * g_regs[h] + b_regs[h])

                for ref, out in ((trows, emb_out), (acc, m_out)):
                    pltpu.make_async_copy(
                        ref.at[pl.ds(a0, CH)],
                        out.at[pl.ds(base + a0, CH)], sem_o).start()

                @pl.when(ci + 2 < n_chunks)
                def _():
                    a2 = pl.multiple_of((ci + 2) * CH, CH)
                    pltpu.make_async_copy(
                        pos_hbm.at[pl.ds(l0 + a2, CH)], pbuf.at[pb],
                        psems.at[pb]).start()

            return 0

        for cp in mcps:
            cp.wait()
        for s in range(min(NBUF, n_steps)):
            fire(s)
        for cp in icps:
            cp.wait()
        g_regs = [gvec[pl.ds(h * LANES, LANES)] for h in range(HV)]
        b_regs = [bvec[pl.ds(h * LANES, LANES)] for h in range(HV)]
        for ci in range(n_chunks):
            pltpu.make_async_copy(
                table_hbm.at[tidx.at[b, pl.ds(ci * CH, CH)]],
                trows.at[pl.ds(ci * CH, CH)], sem_t).start()
        lax.fori_loop(0, n_steps, step_body, 0)

        for ci in range(n_chunks):
            a0 = ci * CH
            for ref, out in ((trows, emb_out), (acc, m_out)):
                pltpu.make_async_copy(
                    ref.at[pl.ds(a0, CH)],
                    out.at[pl.ds(base + a0, CH)], sem_o).wait()

    return k(tgt_ids2d, msa_ids2d, table, pos_tab, gamma, beta)


def kernel(target_ids, input_ids, word_embeddings, position_embeddings, gamma, beta):
    B, L = target_ids.shape
    n_msa = input_ids.shape[1]
    tgt_idx = target_ids.astype(jnp.int32)
    msa_idx = input_ids.astype(jnp.int32).reshape(B * n_msa, L)
    emb, msa_mean = _sc_fused(tgt_idx, msa_idx, word_embeddings,
                              position_embeddings, gamma, beta, B, L, n_msa)
    return emb.reshape(B, L, H), msa_mean.reshape(B, L, H)


# speedup vs baseline: 1.0111x; 1.0111x over previous
import functools
import jax
import jax.numpy as jnp
from jax import lax
from jax.experimental import pallas as pl
from jax.experimental.pallas import tpu as pltpu
from jax.experimental.pallas import tpu_sc as plsc

H = 128
LANES = 16
HV = H // LANES
NC = 2
NS = 16
NW = NC * NS
CH = 64
NBUF = 4
EPS = 1e-12


def _rsqrt(v):
    bits = lax.bitcast_convert_type(v, jnp.int32)
    y = lax.bitcast_convert_type(
        jnp.int32(0x5F3759DF) - lax.shift_right_logical(bits, 1), jnp.float32)
    half = 0.5 * v
    for _ in range(3):
        y = y * (1.5 - half * y * y)
    return y


_GATHER_DNUMS = lax.GatherDimensionNumbers(
    offset_dims=(), collapsed_slice_dims=(0,), start_index_map=(0,))


def _lane_shuffle(x, idx):
    return lax.gather(x, idx[:, None], _GATHER_DNUMS, (1,),
                      mode=lax.GatherScatterMode.PROMISE_IN_BOUNDS)


def _hsum(x):
    lanes = lax.iota(jnp.int32, LANES)
    for sh in (8, 4, 2, 1):
        x = x + _lane_shuffle(x, lanes ^ sh)
    return x


def _sc_fused(tgt_ids2d, msa_ids2d, table, pos_tab, gamma, beta, B, L, n_msa):
    total = B * L
    P = total // NW
    n_chunks = P // CH
    n_steps = n_chunks * n_msa
    inv_n = 1.0 / n_msa
    inv_h = 1.0 / H

    mesh = plsc.VectorSubcoreMesh(core_axis_name="c", subcore_axis_name="s")

    @functools.partial(
        pl.kernel,
        out_type=(
            jax.ShapeDtypeStruct((total, H), jnp.float32),
            jax.ShapeDtypeStruct((total, H), jnp.float32),
        ),
        mesh=mesh,
        scratch_types=[
            pltpu.VMEM((B, P), jnp.int32),
            pltpu.VMEM((P // 128, n_msa, 128), jnp.int32),
            pltpu.VMEM((P, H), jnp.float32),
            pltpu.VMEM((NBUF, CH, H), jnp.float32),
            pltpu.VMEM((P, H), jnp.float32),
            pltpu.VMEM((2, CH, H), jnp.float32),
            pltpu.VMEM((H,), jnp.float32),
            pltpu.VMEM((H,), jnp.float32),
            pltpu.SemaphoreType.DMA,
            pltpu.SemaphoreType.DMA,
            pltpu.SemaphoreType.DMA,
            pltpu.SemaphoreType.DMA,
            pltpu.SemaphoreType.DMA((NBUF,)),
            pltpu.SemaphoreType.DMA((2,)),
            pltpu.SemaphoreType.DMA,
        ],
    )
    def k(tgt_hbm, msa_hbm, table_hbm, pos_hbm, gamma_hbm, beta_hbm,
          emb_out, m_out,
          tidx, midx, trows, ring, acc, pbuf, gvec, bvec,
          sem_im, sem_i, sem_t, sem_a, rsems, psems, sem_o):
        wid = lax.axis_index("s") * NC + lax.axis_index("c")
        base = pl.multiple_of(wid * P, P)
        b = base // L
        l0 = pl.multiple_of(base - b * L, P)

        mcps = []
        for t in range(P // 128):
            mcps.append(pltpu.make_async_copy(
                msa_hbm.at[pl.ds(b * n_msa, n_msa), pl.ds(l0 + t * 128, 128)],
                midx.at[t], sem_im))
        for cp in mcps:
            cp.start()
        icps = [
            pltpu.make_async_copy(tgt_hbm.at[:, pl.ds(l0, P)], tidx, sem_i),
            pltpu.make_async_copy(gamma_hbm, gvec, sem_i),
            pltpu.make_async_copy(beta_hbm, bvec, sem_i),
        ]
        for cp in icps:
            cp.start()

        for ci in range(min(2, n_chunks)):
            pltpu.make_async_copy(
                pos_hbm.at[pl.ds(l0 + ci * CH, CH)], pbuf.at[ci % 2],
                psems.at[ci % 2]).start()

        def _mo(x, m):
            return x if isinstance(x, int) else pl.multiple_of(x, m)

        def gather_cp(step):
            ci = step // n_msa
            j = step - ci * n_msa
            t = ci // 2
            half = ci - t * 2
            isl = midx.at[t, j, pl.ds(_mo(half * CH, CH), CH)]
            a0 = _mo(ci * CH, CH)

            def to_acc():
                return pltpu.make_async_copy(
                    table_hbm.at[isl], acc.at[pl.ds(a0, CH)], sem_a)

            def to_ring():
                sl = step % NBUF
                return pltpu.make_async_copy(
                    table_hbm.at[isl], ring.at[sl], rsems.at[sl])

            return j, ci, to_acc, to_ring

        def fire(step):
            j, _, to_acc, to_ring = gather_cp(step)
            if isinstance(step, int):
                (to_acc() if j == 0 else to_ring()).start()
                return

            @pl.when(j == 0)
            def _():
                to_acc().start()

            @pl.when(j != 0)
            def _():
                to_ring().start()

        def step_body(step, _):
            j, ci, to_acc, to_ring = gather_cp(step)
            a0 = _mo(ci * CH, CH)

            @pl.when(j == 0)
            def _():
                to_acc().wait()

            @pl.when(j != 0)
            def _():
                to_ring().wait()
                sl = step % NBUF
                src = ring.at[sl]

                @plsc.parallel_loop(0, CH, unroll=2)
                def add_row(p):
                    for h in range(HV):
                        hs = pl.ds(h * LANES, LANES)
                        plsc.addupdate(acc.at[a0 + p, hs], src[p, hs])

            @pl.when(step + NBUF < n_steps)
            def _():
                fire(step + NBUF)

            @pl.when(j == n_msa - 1)
            def _():
                pltpu.make_async_copy(
                    table_hbm.at[tidx.at[b, pl.ds(a0, CH)]],
                    trows.at[pl.ds(a0, CH)], sem_t).wait()
                pb = ci % 2
                pltpu.make_async_copy(
                    pos_hbm.at[pl.ds(l0 + a0, CH)], pbuf.at[pb],
                    psems.at[pb]).wait()

                @plsc.parallel_loop(0, CH, unroll=2)
                def fin_row(p):
                    r = a0 + p
                    s16 = jnp.zeros((LANES,), jnp.float32)
                    q16 = jnp.zeros((LANES,), jnp.float32)
                    for h in range(HV):
                        hs = pl.ds(h * LANES, LANES)
                        m = acc[r, hs] * inv_n
                        acc[r, hs] = m
                        x = trows[r, hs] + m + pbuf[pb, p, hs]
                        trows[r, hs] = x
                        s16 = s16 + x
                        q16 = q16 + x * x
                    mean = _hsum(s16) * inv_h
                    var = _hsum(q16) * inv_h - mean * mean
                    inv = _rsqrt(var + EPS)
                    for h in range(HV):
                        hs = pl.ds(h * LANES, LANES)
                        trows[r, hs] = ((trows[r, hs] - mean) * inv
                                        * g_regs[h] + b_regs[h])

                for ref, out in ((trows, emb_out), (acc, m_out)):
                    pltpu.make_async_copy(
                        ref.at[pl.ds(a0, CH)],
                        out.at[pl.ds(base + a0, CH)], sem_o).start()

                @pl.when(ci + 2 < n_chunks)
                def _():
                    a2 = pl.multiple_of((ci + 2) * CH, CH)
                    pltpu.make_async_copy(
                        pos_hbm.at[pl.ds(l0 + a2, CH)], pbuf.at[pb],
                        psems.at[pb]).start()

            return 0

        for cp in mcps:
            cp.wait()
        for s in range(min(NBUF, n_steps)):
            fire(s)
        for cp in icps:
            cp.wait()
        g_regs = [gvec[pl.ds(h * LANES, LANES)] for h in range(HV)]
        b_regs = [bvec[pl.ds(h * LANES, LANES)] for h in range(HV)]
        for ci in range(n_chunks):
            pltpu.make_async_copy(
                table_hbm.at[tidx.at[b, pl.ds(ci * CH, CH)]],
                trows.at[pl.ds(ci * CH, CH)], sem_t).start()
        lax.fori_loop(0, n_steps, step_body, 0)

        for ci in range(n_chunks):
            a0 = ci * CH
            for ref, out in ((trows, emb_out), (acc, m_out)):
                pltpu.make_async_copy(
                    ref.at[pl.ds(a0, CH)],
                    out.at[pl.ds(base + a0, CH)], sem_o).wait()

    return k(tgt_ids2d, msa_ids2d, table, pos_tab, gamma, beta)


def kernel(target_ids, input_ids, word_embeddings, position_embeddings, gamma, beta):
    B, L = target_ids.shape
    n_msa = input_ids.shape[1]
    tgt_idx = target_ids.astype(jnp.int32)
    msa_idx = input_ids.astype(jnp.int32).reshape(B * n_msa, L)
    emb, msa_mean = _sc_fused(tgt_idx, msa_idx, word_embeddings,
                              position_embeddings, gamma, beta, B, L, n_msa)
    return emb.reshape(B, L, H), msa_mean.reshape(B, L, H)
